# initial kernel scaffold (unmeasured)
import jax
import jax.numpy as jnp
from jax import lax
from jax.experimental import pallas as pl
from jax.experimental.pallas import tpu as pltpu

N_DEV = 8
B, SQ, E = 2, 512, 768
HQ, DH = 8, 64
SKV_LOC = 512
NEG = -1e9

_sem_signal = getattr(pl, "semaphore_signal", None) or pltpu.semaphore_signal
_sem_wait = getattr(pl, "semaphore_wait", None) or pltpu.semaphore_wait


def kernel(x, Wq, K_ext, V_ext, Wo):
    def body(x_ref, wq_ref, k_ref, v_ref, wo_ref, out_ref,
             o_g, ml_g, send_sems, recv_sems):
        my = lax.axis_index("i")
        left = lax.rem(my - 1 + N_DEV, N_DEV)
        right = lax.rem(my + 1, N_DEV)

        barrier = pltpu.get_barrier_semaphore()
        for nbr in (left, right):
            _sem_signal(barrier, inc=1, device_id=(nbr,),
                        device_id_type=pl.DeviceIdType.MESH)
        _sem_wait(barrier, 2)

        qi = lax.broadcasted_iota(jnp.int32, (SQ, SKV_LOC), 0)
        kj = lax.broadcasted_iota(jnp.int32, (SQ, SKV_LOC), 1)
        kg = kj + my * SKV_LOC
        mask = (jnp.abs(qi - kg) <= 128) | (kg < 32) | (qi < 32)

        wq = wq_ref[...].astype(jnp.bfloat16)
        for b in range(B):
            xb = x_ref[b].astype(jnp.bfloat16)
            qb = lax.dot(xb, wq, preferred_element_type=jnp.float32)
            qb = qb.astype(jnp.bfloat16)
            kb = k_ref[b].astype(jnp.bfloat16)
            vb = v_ref[b].astype(jnp.bfloat16)
            for h in range(HQ):
                q = qb[:, h * DH:(h + 1) * DH]
                k = kb[:, h, :]
                v = vb[:, h, :]
                s = lax.dot_general(
                    q, k, (((1,), (1,)), ((), ())),
                    preferred_element_type=jnp.float32) * 0.125
                s = jnp.where(mask, s, NEG)
                m = jnp.max(s, axis=1)
                w = jnp.exp(s - m[:, None])
                l = jnp.sum(w, axis=1)
                o = lax.dot(w.astype(jnp.bfloat16), v,
                            preferred_element_type=jnp.float32)
                o_g[0, b, h] = o.astype(jnp.bfloat16)
                ml_g[0, b, h, 0] = m
                ml_g[0, b, h, 1] = l

        for h in range(1, N_DEV):
            rd_o = pltpu.make_async_remote_copy(
                src_ref=o_g.at[h - 1], dst_ref=o_g.at[h],
                send_sem=send_sems.at[0, h], recv_sem=recv_sems.at[0, h],
                device_id=(right,), device_id_type=pl.DeviceIdType.MESH)
            rd_ml = pltpu.make_async_remote_copy(
                src_ref=ml_g.at[h - 1], dst_ref=ml_g.at[h],
                send_sem=send_sems.at[1, h], recv_sem=recv_sems.at[1, h],
                device_id=(right,), device_id_type=pl.DeviceIdType.MESH)
            rd_o.start()
            rd_ml.start()
            rd_o.wait()
            rd_ml.wait()

        ms = [ml_g[s, :, :, 0] for s in range(N_DEV)]
        m_tot = ms[0]
        for s in range(1, N_DEV):
            m_tot = jnp.maximum(m_tot, ms[s])
        l_tot = jnp.zeros((B, HQ, SQ), jnp.float32)
        o_tot = jnp.zeros((B, HQ, SQ, DH), jnp.float32)
        for s in range(N_DEV):
            a = jnp.exp(ms[s] - m_tot)
            l_tot = l_tot + a * ml_g[s, :, :, 1]
            o_tot = o_tot + a[..., None] * o_g[s].astype(jnp.float32)
        ctx = (o_tot / l_tot[..., None]).astype(jnp.bfloat16)

        for b in range(B):
            acc = jnp.zeros((SQ, E), jnp.float32)
            for h in range(HQ):
                wo_h = wo_ref[h * DH:(h + 1) * DH, :].astype(jnp.bfloat16)
                acc = acc + lax.dot(ctx[b, h], wo_h,
                                    preferred_element_type=jnp.float32)
            out_ref[b] = acc

    return pl.pallas_call(
        body,
        out_shape=jax.ShapeDtypeStruct((B, SQ, E), jnp.float32),
        in_specs=[pl.BlockSpec(memory_space=pltpu.VMEM)] * 5,
        out_specs=pl.BlockSpec(memory_space=pltpu.VMEM),
        scratch_shapes=[
            pltpu.VMEM((N_DEV, B, HQ, SQ, DH), jnp.bfloat16),
            pltpu.VMEM((N_DEV, B, HQ, 2, SQ), jnp.float32),
            pltpu.SemaphoreType.DMA((2, N_DEV)),
            pltpu.SemaphoreType.DMA((2, N_DEV)),
        ],
        compiler_params=pltpu.CompilerParams(collective_id=0),
    )(x, Wq, K_ext, V_ext, Wo)


# baseline (device time: 231611 ns/iter reference)
import jax
import jax.numpy as jnp
from jax import lax
from jax.experimental import pallas as pl
from jax.experimental.pallas import tpu as pltpu

N_DEV = 8
B, SQ, E = 2, 512, 768
HQ, DH = 8, 64
SKV_LOC = 512
NEG = -1e9

_sem_signal = getattr(pl, "semaphore_signal", None) or pltpu.semaphore_signal
_sem_wait = getattr(pl, "semaphore_wait", None) or pltpu.semaphore_wait


def kernel(x, Wq, K_ext, V_ext, Wo):
    def body(x_ref, wq_ref, k_ref, v_ref, wo_ref, out_ref,
             o_g, ml_g, send_sems, recv_sems):
        my = lax.axis_index("i")
        left = lax.rem(my - 1 + N_DEV, N_DEV)
        right = lax.rem(my + 1, N_DEV)

        barrier = pltpu.get_barrier_semaphore()
        for nbr in (left, right):
            _sem_signal(barrier, inc=1, device_id=(nbr,),
                        device_id_type=pl.DeviceIdType.MESH)
        _sem_wait(barrier, 2)

        qi = lax.broadcasted_iota(jnp.int32, (SQ, SKV_LOC), 0)
        kj = lax.broadcasted_iota(jnp.int32, (SQ, SKV_LOC), 1)
        kg = kj + my * SKV_LOC
        mask = (jnp.abs(qi - kg) <= 128) | (kg < 32) | (qi < 32)

        wq = wq_ref[...].astype(jnp.bfloat16)
        for b in range(B):
            xb = x_ref[b].astype(jnp.bfloat16)
            qb = lax.dot(xb, wq, preferred_element_type=jnp.float32)
            qb = qb.astype(jnp.bfloat16)
            kb = k_ref[b].astype(jnp.bfloat16)
            vb = v_ref[b].astype(jnp.bfloat16)
            for h in range(HQ):
                q = qb[:, h * DH:(h + 1) * DH]
                k = kb[:, h, :]
                v = vb[:, h, :]
                s = lax.dot_general(
                    q, k, (((1,), (1,)), ((), ())),
                    preferred_element_type=jnp.float32) * 0.125
                s = jnp.where(mask, s, NEG)
                m = jnp.max(s, axis=1)
                w = jnp.exp(s - m[:, None])
                l = jnp.sum(w, axis=1)
                o = lax.dot(w.astype(jnp.bfloat16), v,
                            preferred_element_type=jnp.float32)
                o_g[0, b, h] = o.astype(jnp.bfloat16)
                ml_g[0, b, h, 0] = m
                ml_g[0, b, h, 1] = l

        for h in range(1, N_DEV):
            rd_o = pltpu.make_async_remote_copy(
                src_ref=o_g.at[h - 1], dst_ref=o_g.at[h],
                send_sem=send_sems.at[0, h], recv_sem=recv_sems.at[0, h],
                device_id=(right,), device_id_type=pl.DeviceIdType.MESH)
            rd_ml = pltpu.make_async_remote_copy(
                src_ref=ml_g.at[h - 1], dst_ref=ml_g.at[h],
                send_sem=send_sems.at[1, h], recv_sem=recv_sems.at[1, h],
                device_id=(right,), device_id_type=pl.DeviceIdType.MESH)
            rd_o.start()
            rd_ml.start()
            rd_o.wait()
            rd_ml.wait()

        ms = [ml_g[s, :, :, 0] for s in range(N_DEV)]
        m_tot = ms[0]
        for s in range(1, N_DEV):
            m_tot = jnp.maximum(m_tot, ms[s])
        l_tot = jnp.zeros((B, HQ, SQ), jnp.float32)
        o_tot = jnp.zeros((B, HQ, SQ, DH), jnp.float32)
        for s in range(N_DEV):
            a = jnp.exp(ms[s] - m_tot)
            l_tot = l_tot + a * ml_g[s, :, :, 1]
            o_tot = o_tot + a[..., None] * o_g[s].astype(jnp.float32)
        ctx = (o_tot / l_tot[..., None]).astype(jnp.bfloat16)

        for b in range(B):
            acc = jnp.zeros((SQ, E), jnp.float32)
            for h in range(HQ):
                wo_h = wo_ref[h * DH:(h + 1) * DH, :].astype(jnp.bfloat16)
                acc = acc + lax.dot(ctx[b, h], wo_h,
                                    preferred_element_type=jnp.float32)
            out_ref[b] = acc

    return pl.pallas_call(
        body,
        out_shape=jax.ShapeDtypeStruct((B, SQ, E), jnp.float32),
        in_specs=[pl.BlockSpec(memory_space=pltpu.VMEM)] * 5,
        out_specs=pl.BlockSpec(memory_space=pltpu.VMEM),
        scratch_shapes=[
            pltpu.VMEM((N_DEV, B, HQ, SQ, DH), jnp.bfloat16),
            pltpu.VMEM((N_DEV, B, HQ, 2, SQ), jnp.float32),
            pltpu.SemaphoreType.DMA((2, N_DEV)),
            pltpu.SemaphoreType.DMA((2, N_DEV)),
        ],
        compiler_params=pltpu.CompilerParams(
            collective_id=0, vmem_limit_bytes=64 * 1024 * 1024),
    )(x, Wq, K_ext, V_ext, Wo)


# device time: 131059 ns/iter; 1.7672x vs baseline; 1.7672x over previous
import jax
import jax.numpy as jnp
from jax import lax
from jax.experimental import pallas as pl
from jax.experimental.pallas import tpu as pltpu

N_DEV = 8
N_STEPS = 3
B, SQ, E = 2, 512, 768
HQ, DH = 8, 64
SKV_LOC = 512
NEG = -1e9

_sem_signal = getattr(pl, "semaphore_signal", None) or pltpu.semaphore_signal
_sem_wait = getattr(pl, "semaphore_wait", None) or pltpu.semaphore_wait


def kernel(x, Wq, K_ext, V_ext, Wo):
    def body(x_ref, wq_ref, k_ref, v_ref, wo_ref, out_ref,
             acc_o, acc_ml, rcv_o, rcv_ml, send_sems, recv_sems):
        my = lax.axis_index("i")
        partners = [jnp.bitwise_xor(my, 1 << k) for k in range(N_STEPS)]

        barrier = pltpu.get_barrier_semaphore()
        for p in partners:
            _sem_signal(barrier, inc=1, device_id=(p,),
                        device_id_type=pl.DeviceIdType.MESH)
        _sem_wait(barrier, N_STEPS)

        qi = lax.broadcasted_iota(jnp.int32, (SQ, SKV_LOC), 0)
        kj = lax.broadcasted_iota(jnp.int32, (SQ, SKV_LOC), 1)
        kg = kj + my * SKV_LOC
        mask = (jnp.abs(qi - kg) <= 128) | (kg < 32) | (qi < 32)

        wq = wq_ref[...].astype(jnp.bfloat16)
        for b in range(B):
            xb = x_ref[b].astype(jnp.bfloat16)
            qb = lax.dot(xb, wq, preferred_element_type=jnp.float32)
            qb = qb.astype(jnp.bfloat16)
            kb = k_ref[b].astype(jnp.bfloat16)
            vb = v_ref[b].astype(jnp.bfloat16)
            for h in range(HQ):
                q = qb[:, h * DH:(h + 1) * DH]
                k = kb[:, h, :]
                v = vb[:, h, :]
                s = lax.dot_general(
                    q, k, (((1,), (1,)), ((), ())),
                    preferred_element_type=jnp.float32) * 0.125
                s = jnp.where(mask, s, NEG)
                m = jnp.max(s, axis=1)
                w = jnp.exp(s - m[:, None])
                l = jnp.sum(w, axis=1)
                o = lax.dot(w.astype(jnp.bfloat16), v,
                            preferred_element_type=jnp.float32)
                acc_o[b, h] = o.astype(jnp.bfloat16)
                acc_ml[b, h, 0] = m
                acc_ml[b, h, 1] = l

        for k in range(N_STEPS):
            rd_o = pltpu.make_async_remote_copy(
                src_ref=acc_o, dst_ref=rcv_o.at[k],
                send_sem=send_sems.at[0, k], recv_sem=recv_sems.at[0, k],
                device_id=(partners[k],), device_id_type=pl.DeviceIdType.MESH)
            rd_ml = pltpu.make_async_remote_copy(
                src_ref=acc_ml, dst_ref=rcv_ml.at[k],
                send_sem=send_sems.at[1, k], recv_sem=recv_sems.at[1, k],
                device_id=(partners[k],), device_id_type=pl.DeviceIdType.MESH)
            rd_o.start()
            rd_ml.start()
            rd_o.wait()
            rd_ml.wait()

            m_s = acc_ml[:, :, 0]
            l_s = acc_ml[:, :, 1]
            m_r = rcv_ml[k, :, :, 0]
            l_r = rcv_ml[k, :, :, 1]
            m_n = jnp.maximum(m_s, m_r)
            a_s = jnp.exp(m_s - m_n)
            a_r = jnp.exp(m_r - m_n)
            acc_ml[:, :, 0] = m_n
            acc_ml[:, :, 1] = a_s * l_s + a_r * l_r
            o_n = (a_s[..., None] * acc_o[...].astype(jnp.float32)
                   + a_r[..., None] * rcv_o[k].astype(jnp.float32))
            acc_o[...] = o_n.astype(jnp.bfloat16)

        ctx = (acc_o[...].astype(jnp.float32)
               / acc_ml[:, :, 1][..., None]).astype(jnp.bfloat16)
        for b in range(B):
            acc = jnp.zeros((SQ, E), jnp.float32)
            for h in range(HQ):
                wo_h = wo_ref[h * DH:(h + 1) * DH, :].astype(jnp.bfloat16)
                acc = acc + lax.dot(ctx[b, h], wo_h,
                                    preferred_element_type=jnp.float32)
            out_ref[b] = acc

    return pl.pallas_call(
        body,
        out_shape=jax.ShapeDtypeStruct((B, SQ, E), jnp.float32),
        in_specs=[pl.BlockSpec(memory_space=pltpu.VMEM)] * 5,
        out_specs=pl.BlockSpec(memory_space=pltpu.VMEM),
        scratch_shapes=[
            pltpu.VMEM((B, HQ, SQ, DH), jnp.bfloat16),
            pltpu.VMEM((B, HQ, 2, SQ), jnp.float32),
            pltpu.VMEM((N_STEPS, B, HQ, SQ, DH), jnp.bfloat16),
            pltpu.VMEM((N_STEPS, B, HQ, 2, SQ), jnp.float32),
            pltpu.SemaphoreType.DMA((2, N_STEPS)),
            pltpu.SemaphoreType.DMA((2, N_STEPS)),
        ],
        compiler_params=pltpu.CompilerParams(
            collective_id=0, vmem_limit_bytes=64 * 1024 * 1024),
    )(x, Wq, K_ext, V_ext, Wo)
